# Initial kernel scaffold; baseline (speedup 1.0000x reference)
import jax
import jax.numpy as jnp
from jax.experimental import pallas as pl


def kernel(down_mask, up_mask):
    raise NotImplementedError("write your pallas kernel here")


# speedup vs baseline: 73.5722x; 73.5722x over previous
import functools

import jax
import jax.numpy as jnp
from jax import lax
from jax.experimental import pallas as pl
from jax.experimental.pallas import tpu as pltpu
from jax.experimental.pallas import tpu_sc as plsc

N = 4096 * 1024
_SPARS = 0.1
KTOP = N - int((1.0 - _SPARS) * N)

NS = 16
PER_TILE = N // NS
C = 8192
NCH = PER_TILE // C
NBUF = 4

NB1, NB2, NB3 = 2048, 1024, 1024
SH1, SH2 = 20, 10

_f32 = jnp.float32
_i32 = jnp.int32


_ONE_F32_BITS = 0x3F800000


def _absbits(v):
    return jnp.bitwise_and(v, jnp.int32(0x7FFFFFFF))


def _run(in_hbm, out_hbm, s, h1, h2, h3, lh, bufs, sp, isems, osems):
    base = s * PER_TILE
    iota = lax.iota(_i32, 16)
    ones16 = jnp.full((16,), 1, _i32)
    zeros16 = jnp.full((16,), 0, _i32)

    def _zero(h, nb):
        def zb(g, carry):
            for l in range(16):
                h[l, pl.ds(g * 16, 16)] = zeros16
            return carry
        lax.fori_loop(0, nb // 16, zb, 0)

    _zero(h1, NB1)
    _zero(h2, NB2)
    _zero(h3, NB3)

    def _start_in(ch, buf, sem):
        pltpu.async_copy(in_hbm.at[pl.ds(base + ch * C, C)], buf, sem)

    def _wait_in(buf, sem):
        pltpu.make_async_copy(in_hbm.at[pl.ds(base, C)], buf, sem).wait()

    def stream_pass(process):
        for b in range(NBUF):
            _start_in(b, bufs[b], isems[b])

        def body(p, carry):
            for b in range(NBUF):
                ch = NBUF * p + b
                _wait_in(bufs[b], isems[b])

                def inner(i, c2, _buf=bufs[b]):
                    process(_buf[pl.ds(i * 16, 16)])
                    return c2
                lax.fori_loop(0, C // 16, inner, 0)
                nxt = ch + NBUF

                @pl.when(nxt < NCH)
                def _():
                    _start_in(nxt, bufs[b], isems[b])
            return carry
        lax.fori_loop(0, NCH // NBUF, body, 0)

    def merge_and_scan(h, nb, kp):
        def lr(g, carry):
            acc = h[0, pl.ds(g * 16, 16)]
            for l in range(1, 16):
                acc = acc + h[l, pl.ds(g * 16, 16)]
            lh[pl.ds(g * 16, 16)] = acc
            return carry
        lax.fori_loop(0, nb // 16, lr, 0)

        pltpu.sync_copy(lh, sp.at[s])
        plsc.subcore_barrier()
        pltpu.sync_copy(sp, h1)
        plsc.subcore_barrier()

        ng = nb // 16

        def scan_body(t, carry):
            cnt, maxfail, above = carry
            g = ng - 1 - t
            m = h1[0, pl.ds(g * 16, 16)]
            for r in range(1, 16):
                m = m + h1[r, pl.ds(g * 16, 16)]
            sfx = jnp.flip(plsc.cumsum(jnp.flip(m))) + above
            ge = sfx >= kp
            cnt = cnt + jnp.sum(jnp.where(ge, 1, 0))
            maxfail = jnp.maximum(maxfail, jnp.max(jnp.where(ge, 0, sfx)))
            above = above + jnp.sum(m)
            return cnt, maxfail, above

        cnt, maxfail, _ = lax.fori_loop(
            0, ng, scan_body, (jnp.int32(0), jnp.int32(0), jnp.int32(0)))
        return cnt - 1, kp - maxfail

    def p1(v):
        bits = _absbits(v)
        plsc.addupdate_scatter(h1, [iota, bits >> SH1], ones16)
    stream_pass(p1)
    b1r, k2 = merge_and_scan(h1, NB1, jnp.int32(KTOP))

    def p2(v):
        bits = _absbits(v)
        pred = (bits >> SH1) == b1r
        binv = jnp.bitwise_and(bits >> SH2, jnp.int32(NB2 - 1))
        plsc.addupdate_scatter(h2, [iota, binv], ones16, mask=pred)
    stream_pass(p2)
    b2r, k3 = merge_and_scan(h2, NB2, k2)
    pfx2 = (b1r << 10) | b2r

    def p3(v):
        bits = _absbits(v)
        pred = (bits >> SH2) == pfx2
        binv = jnp.bitwise_and(bits, jnp.int32(NB3 - 1))
        plsc.addupdate_scatter(h3, [iota, binv], ones16, mask=pred)
    stream_pass(p3)
    b3r, _ = merge_and_scan(h3, NB3, k3)
    thr = (pfx2 << 10) | b3r

    def _start_out(ch, buf, sem):
        pltpu.async_copy(buf, out_hbm.at[pl.ds(base + ch * C, C)], sem)

    def _wait_out(buf, sem):
        pltpu.make_async_copy(buf, out_hbm.at[pl.ds(base, C)], sem).wait()

    for b in range(NBUF):
        _start_in(b, bufs[b], isems[b])

    def mbody(p, carry):
        for b in range(NBUF):
            ch = NBUF * p + b
            _wait_in(bufs[b], isems[b])

            def inner(i, c2, _buf=bufs[b]):
                bits = _absbits(_buf[pl.ds(i * 16, 16)])
                _buf[pl.ds(i * 16, 16)] = jnp.where(
                    bits >= thr, jnp.int32(_ONE_F32_BITS), jnp.int32(0))
                return c2
            lax.fori_loop(0, C // 16, inner, 0)
            _start_out(ch, bufs[b], osems[b])
            nxt = ch + NBUF

            @pl.when(nxt < NCH)
            def _():
                _wait_out(bufs[b], osems[b])
                _start_in(nxt, bufs[b], isems[b])
        return carry
    lax.fori_loop(0, NCH // NBUF, mbody, 0)
    for b in range(NBUF):
        _wait_out(bufs[b], osems[b])


def _body(down_hbm, up_hbm, dout_hbm, uout_hbm,
          h1, h2, h3, lh, b0, b1, b2, b3, sp,
          i0, i1, i2, i3, o0, o1, o2, o3):
    c = lax.axis_index("c")
    s = lax.axis_index("s")
    bufs = (b0, b1, b2, b3)
    isems = (i0, i1, i2, i3)
    osems = (o0, o1, o2, o3)

    @pl.when(c == 0)
    def _():
        _run(down_hbm, dout_hbm, s, h1, h2, h3, lh, bufs, sp, isems, osems)

    @pl.when(c == 1)
    def _():
        _run(up_hbm, uout_hbm, s, h1, h2, h3, lh, bufs, sp, isems, osems)


_mesh = plsc.VectorSubcoreMesh(core_axis_name="c", subcore_axis_name="s")

_select = functools.partial(
    pl.kernel,
    out_type=(jax.ShapeDtypeStruct((N,), _i32),
              jax.ShapeDtypeStruct((N,), _i32)),
    mesh=_mesh,
    scratch_types=[
        pltpu.VMEM((16, NB1), _i32),
        pltpu.VMEM((16, NB2), _i32),
        pltpu.VMEM((16, NB3), _i32),
        pltpu.VMEM((NB1,), _i32),
        pltpu.VMEM((C,), _i32),
        pltpu.VMEM((C,), _i32),
        pltpu.VMEM((C,), _i32),
        pltpu.VMEM((C,), _i32),
        pltpu.VMEM_SHARED((16, NB1), _i32),
        pltpu.SemaphoreType.DMA,
        pltpu.SemaphoreType.DMA,
        pltpu.SemaphoreType.DMA,
        pltpu.SemaphoreType.DMA,
        pltpu.SemaphoreType.DMA,
        pltpu.SemaphoreType.DMA,
        pltpu.SemaphoreType.DMA,
        pltpu.SemaphoreType.DMA,
    ],
    compiler_params=pltpu.CompilerParams(
        use_tc_tiling_on_sc=False, needs_layout_passes=False),
)(_body)


@jax.jit
def kernel(down_mask, up_mask):
    db = lax.bitcast_convert_type(down_mask.reshape(-1), _i32)
    ub = lax.bitcast_convert_type(up_mask.reshape(-1), _i32)
    d, u = _select(db, ub)
    return (lax.bitcast_convert_type(d, _f32).reshape(down_mask.shape),
            lax.bitcast_convert_type(u, _f32).reshape(up_mask.shape))


# speedup vs baseline: 86.8477x; 1.1804x over previous
import functools

import jax
import jax.numpy as jnp
from jax import lax
from jax.experimental import pallas as pl
from jax.experimental.pallas import tpu as pltpu
from jax.experimental.pallas import tpu_sc as plsc

N = 4096 * 1024
_SPARS = 0.1
KTOP = N - int((1.0 - _SPARS) * N)

NS = 16
PER_TILE = N // NS
C = 8192
NCH = PER_TILE // C
NBUF = 4
UNROLL = 8

NB1, NB2, NB3 = 2048, 1024, 1024
SH1, SH2 = 20, 10

_f32 = jnp.float32
_i32 = jnp.int32


_ONE_F32_BITS = 0x3F800000


def _absbits(v):
    return jnp.bitwise_and(v, jnp.int32(0x7FFFFFFF))


def _run(in_hbm, out_hbm, s, h1, h2, h3, lh, bufs, sp, isems, osems):
    base = s * PER_TILE
    iota = lax.iota(_i32, 16)
    ones16 = jnp.full((16,), 1, _i32)
    zeros16 = jnp.full((16,), 0, _i32)

    def _zero(h, nb):
        def zb(g, carry):
            for l in range(16):
                h[l, pl.ds(g * 16, 16)] = zeros16
            return carry
        lax.fori_loop(0, nb // 16, zb, 0)

    _zero(h1, NB1)
    _zero(h2, NB2)
    _zero(h3, NB3)

    def _start_in(ch, buf, sem):
        pltpu.async_copy(in_hbm.at[pl.ds(base + ch * C, C)], buf, sem)

    def _wait_in(buf, sem):
        pltpu.make_async_copy(in_hbm.at[pl.ds(base, C)], buf, sem).wait()

    def stream_pass(process):
        for b in range(NBUF):
            _start_in(b, bufs[b], isems[b])

        def body(p, carry):
            for b in range(NBUF):
                ch = NBUF * p + b
                _wait_in(bufs[b], isems[b])

                def inner(i, c2, _buf=bufs[b]):
                    for u in range(UNROLL):
                        process(_buf[pl.ds(i * (16 * UNROLL) + u * 16, 16)])
                    return c2
                lax.fori_loop(0, C // (16 * UNROLL), inner, 0)
                nxt = ch + NBUF

                @pl.when(nxt < NCH)
                def _():
                    _start_in(nxt, bufs[b], isems[b])
            return carry
        lax.fori_loop(0, NCH // NBUF, body, 0)

    def merge_and_scan(h, nb, kp):
        def lr(g, carry):
            acc = h[0, pl.ds(g * 16, 16)]
            for l in range(1, 16):
                acc = acc + h[l, pl.ds(g * 16, 16)]
            lh[pl.ds(g * 16, 16)] = acc
            return carry
        lax.fori_loop(0, nb // 16, lr, 0)

        pltpu.sync_copy(lh, sp.at[s])
        plsc.subcore_barrier()
        pltpu.sync_copy(sp, h1)
        plsc.subcore_barrier()

        ng = nb // 16

        def scan_body(t, carry):
            cnt, maxfail, above = carry
            g = ng - 1 - t
            m = h1[0, pl.ds(g * 16, 16)]
            for r in range(1, 16):
                m = m + h1[r, pl.ds(g * 16, 16)]
            sfx = jnp.flip(plsc.cumsum(jnp.flip(m))) + above
            ge = sfx >= kp
            cnt = cnt + jnp.sum(jnp.where(ge, 1, 0))
            maxfail = jnp.maximum(maxfail, jnp.max(jnp.where(ge, 0, sfx)))
            above = above + jnp.sum(m)
            return cnt, maxfail, above

        cnt, maxfail, _ = lax.fori_loop(
            0, ng, scan_body, (jnp.int32(0), jnp.int32(0), jnp.int32(0)))
        return cnt - 1, kp - maxfail

    def p1(v):
        bits = _absbits(v)
        plsc.addupdate_scatter(h1, [iota, bits >> SH1], ones16)
    stream_pass(p1)
    b1r, k2 = merge_and_scan(h1, NB1, jnp.int32(KTOP))

    def p2(v):
        bits = _absbits(v)
        pred = (bits >> SH1) == b1r
        binv = jnp.bitwise_and(bits >> SH2, jnp.int32(NB2 - 1))
        plsc.addupdate_scatter(h2, [iota, binv], ones16, mask=pred)
    stream_pass(p2)
    b2r, k3 = merge_and_scan(h2, NB2, k2)
    pfx2 = (b1r << 10) | b2r

    def p3(v):
        bits = _absbits(v)
        pred = (bits >> SH2) == pfx2
        binv = jnp.bitwise_and(bits, jnp.int32(NB3 - 1))
        plsc.addupdate_scatter(h3, [iota, binv], ones16, mask=pred)
    stream_pass(p3)
    b3r, _ = merge_and_scan(h3, NB3, k3)
    thr = (pfx2 << 10) | b3r

    def _start_out(ch, buf, sem):
        pltpu.async_copy(buf, out_hbm.at[pl.ds(base + ch * C, C)], sem)

    def _wait_out(buf, sem):
        pltpu.make_async_copy(buf, out_hbm.at[pl.ds(base, C)], sem).wait()

    for b in range(NBUF):
        _start_in(b, bufs[b], isems[b])

    def mbody(p, carry):
        for b in range(NBUF):
            ch = NBUF * p + b
            _wait_in(bufs[b], isems[b])

            def inner(i, c2, _buf=bufs[b]):
                for u in range(UNROLL):
                    off = i * (16 * UNROLL) + u * 16
                    bits = _absbits(_buf[pl.ds(off, 16)])
                    _buf[pl.ds(off, 16)] = jnp.where(
                        bits >= thr, jnp.int32(_ONE_F32_BITS), jnp.int32(0))
                return c2
            lax.fori_loop(0, C // (16 * UNROLL), inner, 0)
            _start_out(ch, bufs[b], osems[b])
            nxt = ch + NBUF

            @pl.when(nxt < NCH)
            def _():
                _wait_out(bufs[b], osems[b])
                _start_in(nxt, bufs[b], isems[b])
        return carry
    lax.fori_loop(0, NCH // NBUF, mbody, 0)
    for b in range(NBUF):
        _wait_out(bufs[b], osems[b])


def _body(down_hbm, up_hbm, dout_hbm, uout_hbm,
          h1, h2, h3, lh, b0, b1, b2, b3, sp,
          i0, i1, i2, i3, o0, o1, o2, o3):
    c = lax.axis_index("c")
    s = lax.axis_index("s")
    bufs = (b0, b1, b2, b3)
    isems = (i0, i1, i2, i3)
    osems = (o0, o1, o2, o3)

    @pl.when(c == 0)
    def _():
        _run(down_hbm, dout_hbm, s, h1, h2, h3, lh, bufs, sp, isems, osems)

    @pl.when(c == 1)
    def _():
        _run(up_hbm, uout_hbm, s, h1, h2, h3, lh, bufs, sp, isems, osems)


_mesh = plsc.VectorSubcoreMesh(core_axis_name="c", subcore_axis_name="s")

_select = functools.partial(
    pl.kernel,
    out_type=(jax.ShapeDtypeStruct((N,), _i32),
              jax.ShapeDtypeStruct((N,), _i32)),
    mesh=_mesh,
    scratch_types=[
        pltpu.VMEM((16, NB1), _i32),
        pltpu.VMEM((16, NB2), _i32),
        pltpu.VMEM((16, NB3), _i32),
        pltpu.VMEM((NB1,), _i32),
        pltpu.VMEM((C,), _i32),
        pltpu.VMEM((C,), _i32),
        pltpu.VMEM((C,), _i32),
        pltpu.VMEM((C,), _i32),
        pltpu.VMEM_SHARED((16, NB1), _i32),
        pltpu.SemaphoreType.DMA,
        pltpu.SemaphoreType.DMA,
        pltpu.SemaphoreType.DMA,
        pltpu.SemaphoreType.DMA,
        pltpu.SemaphoreType.DMA,
        pltpu.SemaphoreType.DMA,
        pltpu.SemaphoreType.DMA,
        pltpu.SemaphoreType.DMA,
    ],
    compiler_params=pltpu.CompilerParams(
        use_tc_tiling_on_sc=False, needs_layout_passes=False),
)(_body)


@jax.jit
def kernel(down_mask, up_mask):
    db = lax.bitcast_convert_type(down_mask.reshape(-1), _i32)
    ub = lax.bitcast_convert_type(up_mask.reshape(-1), _i32)
    d, u = _select(db, ub)
    return (lax.bitcast_convert_type(d, _f32).reshape(down_mask.shape),
            lax.bitcast_convert_type(u, _f32).reshape(up_mask.shape))


# speedup vs baseline: 209.4162x; 2.4113x over previous
import functools

import jax
import jax.numpy as jnp
from jax import lax
from jax.experimental import pallas as pl
from jax.experimental.pallas import tpu as pltpu
from jax.experimental.pallas import tpu_sc as plsc

N = 4096 * 1024
_SPARS = 0.1
KTOP = N - int((1.0 - _SPARS) * N)

NS = 16
PER_TILE = N // NS
C = 8192
NCH = PER_TILE // C
NBUF = 4
UNROLL = 8

NB1, NB2, NB3 = 2048, 1024, 1024
SH1, SH2 = 20, 10

_f32 = jnp.float32
_i32 = jnp.int32


_ONE_F32_BITS = 0x3F800000


def _absbits(v):
    return jnp.bitwise_and(v, jnp.int32(0x7FFFFFFF))


def _run(in_hbm, out_hbm, s, h1, h2, h3, lh, bufs, sp, isems, osems):
    base = s * PER_TILE
    iota = lax.iota(_i32, 16)
    ones16 = jnp.full((16,), 1, _i32)
    zeros16 = jnp.full((16,), 0, _i32)

    def _zero(h, nb):
        def zb(g, carry):
            for l in range(16):
                h[l, pl.ds(g * 16, 16)] = zeros16
            return carry
        lax.fori_loop(0, nb // 16, zb, 0)

    _zero(h1, NB1)
    _zero(h2, NB2)
    _zero(h3, NB3)

    def _start_in(ch, buf, sem):
        pltpu.async_copy(in_hbm.at[pl.ds(base + ch * C, C)], buf, sem)

    def _wait_in(buf, sem):
        pltpu.make_async_copy(in_hbm.at[pl.ds(base, C)], buf, sem).wait()

    def stream_pass(process):
        for b in range(NBUF):
            _start_in(b, bufs[b], isems[b])

        def body(p, carry):
            for b in range(NBUF):
                ch = NBUF * p + b
                _wait_in(bufs[b], isems[b])

                @plsc.parallel_loop(0, C, 16, unroll=UNROLL)
                def inner(i, _buf=bufs[b]):
                    process(_buf[pl.ds(i, 16)])
                nxt = ch + NBUF

                @pl.when(nxt < NCH)
                def _():
                    _start_in(nxt, bufs[b], isems[b])
            return carry
        lax.fori_loop(0, NCH // NBUF, body, 0)

    def merge_and_scan(h, nb, kp):
        def lr(g, carry):
            acc = h[0, pl.ds(g * 16, 16)]
            for l in range(1, 16):
                acc = acc + h[l, pl.ds(g * 16, 16)]
            lh[pl.ds(g * 16, 16)] = acc
            return carry
        lax.fori_loop(0, nb // 16, lr, 0)

        pltpu.sync_copy(lh, sp.at[s])
        plsc.subcore_barrier()
        pltpu.sync_copy(sp, h1)
        plsc.subcore_barrier()

        ng = nb // 16

        def scan_body(t, carry):
            cnt, maxfail, above = carry
            g = ng - 1 - t
            m = h1[0, pl.ds(g * 16, 16)]
            for r in range(1, 16):
                m = m + h1[r, pl.ds(g * 16, 16)]
            sfx = jnp.flip(plsc.cumsum(jnp.flip(m))) + above
            ge = sfx >= kp
            cnt = cnt + jnp.sum(jnp.where(ge, 1, 0))
            maxfail = jnp.maximum(maxfail, jnp.max(jnp.where(ge, 0, sfx)))
            above = above + jnp.sum(m)
            return cnt, maxfail, above

        cnt, maxfail, _ = lax.fori_loop(
            0, ng, scan_body, (jnp.int32(0), jnp.int32(0), jnp.int32(0)))
        return cnt - 1, kp - maxfail

    def p1(v):
        bits = _absbits(v)
        plsc.addupdate_scatter(h1, [iota, bits >> SH1], ones16)
    stream_pass(p1)
    b1r, k2 = merge_and_scan(h1, NB1, jnp.int32(KTOP))

    def p2(v):
        bits = _absbits(v)
        pred = (bits >> SH1) == b1r
        binv = jnp.bitwise_and(bits >> SH2, jnp.int32(NB2 - 1))
        plsc.addupdate_scatter(h2, [iota, binv], ones16, mask=pred)
    stream_pass(p2)
    b2r, k3 = merge_and_scan(h2, NB2, k2)
    pfx2 = (b1r << 10) | b2r

    def p3(v):
        bits = _absbits(v)
        pred = (bits >> SH2) == pfx2
        binv = jnp.bitwise_and(bits, jnp.int32(NB3 - 1))
        plsc.addupdate_scatter(h3, [iota, binv], ones16, mask=pred)
    stream_pass(p3)
    b3r, _ = merge_and_scan(h3, NB3, k3)
    thr = (pfx2 << 10) | b3r

    def _start_out(ch, buf, sem):
        pltpu.async_copy(buf, out_hbm.at[pl.ds(base + ch * C, C)], sem)

    def _wait_out(buf, sem):
        pltpu.make_async_copy(buf, out_hbm.at[pl.ds(base, C)], sem).wait()

    for b in range(NBUF):
        _start_in(b, bufs[b], isems[b])

    def mbody(p, carry):
        for b in range(NBUF):
            ch = NBUF * p + b
            _wait_in(bufs[b], isems[b])

            @plsc.parallel_loop(0, C, 16, unroll=UNROLL)
            def inner(i, _buf=bufs[b]):
                bits = _absbits(_buf[pl.ds(i, 16)])
                _buf[pl.ds(i, 16)] = jnp.where(
                    bits >= thr, jnp.int32(_ONE_F32_BITS), jnp.int32(0))
            _start_out(ch, bufs[b], osems[b])
            nxt = ch + NBUF

            @pl.when(nxt < NCH)
            def _():
                _wait_out(bufs[b], osems[b])
                _start_in(nxt, bufs[b], isems[b])
        return carry
    lax.fori_loop(0, NCH // NBUF, mbody, 0)
    for b in range(NBUF):
        _wait_out(bufs[b], osems[b])


def _body(down_hbm, up_hbm, dout_hbm, uout_hbm,
          h1, h2, h3, lh, b0, b1, b2, b3, sp,
          i0, i1, i2, i3, o0, o1, o2, o3):
    c = lax.axis_index("c")
    s = lax.axis_index("s")
    bufs = (b0, b1, b2, b3)
    isems = (i0, i1, i2, i3)
    osems = (o0, o1, o2, o3)

    @pl.when(c == 0)
    def _():
        _run(down_hbm, dout_hbm, s, h1, h2, h3, lh, bufs, sp, isems, osems)

    @pl.when(c == 1)
    def _():
        _run(up_hbm, uout_hbm, s, h1, h2, h3, lh, bufs, sp, isems, osems)


_mesh = plsc.VectorSubcoreMesh(core_axis_name="c", subcore_axis_name="s")

_select = functools.partial(
    pl.kernel,
    out_type=(jax.ShapeDtypeStruct((N,), _i32),
              jax.ShapeDtypeStruct((N,), _i32)),
    mesh=_mesh,
    scratch_types=[
        pltpu.VMEM((16, NB1), _i32),
        pltpu.VMEM((16, NB2), _i32),
        pltpu.VMEM((16, NB3), _i32),
        pltpu.VMEM((NB1,), _i32),
        pltpu.VMEM((C,), _i32),
        pltpu.VMEM((C,), _i32),
        pltpu.VMEM((C,), _i32),
        pltpu.VMEM((C,), _i32),
        pltpu.VMEM_SHARED((16, NB1), _i32),
        pltpu.SemaphoreType.DMA,
        pltpu.SemaphoreType.DMA,
        pltpu.SemaphoreType.DMA,
        pltpu.SemaphoreType.DMA,
        pltpu.SemaphoreType.DMA,
        pltpu.SemaphoreType.DMA,
        pltpu.SemaphoreType.DMA,
        pltpu.SemaphoreType.DMA,
    ],
    compiler_params=pltpu.CompilerParams(
        use_tc_tiling_on_sc=False, needs_layout_passes=False),
)(_body)


@jax.jit
def kernel(down_mask, up_mask):
    db = lax.bitcast_convert_type(down_mask.reshape(-1), _i32)
    ub = lax.bitcast_convert_type(up_mask.reshape(-1), _i32)
    d, u = _select(db, ub)
    return (lax.bitcast_convert_type(d, _f32).reshape(down_mask.shape),
            lax.bitcast_convert_type(u, _f32).reshape(up_mask.shape))


# speedup vs baseline: 302.5730x; 1.4448x over previous
import functools

import jax
import jax.numpy as jnp
from jax import lax
from jax.experimental import pallas as pl
from jax.experimental.pallas import tpu as pltpu
from jax.experimental.pallas import tpu_sc as plsc

N = 4096 * 1024
_SPARS = 0.1
KTOP = N - int((1.0 - _SPARS) * N)

NS = 16
PER_TILE = N // NS
C = 8192
NCH = PER_TILE // C
NBUF = 4
UNROLL = 8

NB1, NB2, NB3 = 2048, 1024, 1024
SH1, SH2 = 20, 10

_f32 = jnp.float32
_i32 = jnp.int32


_ONE_F32_BITS = 0x3F800000


def _absbits(v):
    return jnp.bitwise_and(v, jnp.int32(0x7FFFFFFF))


def _run(in_hbm, out_hbm, s, h1, h2, h3, lh, bufs, sp, isems, osems):
    base = s * PER_TILE
    iota = lax.iota(_i32, 16)
    ones16 = jnp.full((16,), 1, _i32)
    zeros16 = jnp.full((16,), 0, _i32)

    def _zero(h, nb):
        def zb(g, carry):
            for l in range(16):
                h[l, pl.ds(g * 16, 16)] = zeros16
            return carry
        lax.fori_loop(0, nb // 16, zb, 0)

    _zero(h1, NB1)
    _zero(h2, NB2)
    _zero(h3, NB3)

    def _start_in(ch, buf, sem):
        pltpu.async_copy(in_hbm.at[pl.ds(base + ch * C, C)], buf, sem)

    def _wait_in(buf, sem):
        pltpu.make_async_copy(in_hbm.at[pl.ds(base, C)], buf, sem).wait()

    def stream_pass(process):
        for b in range(NBUF):
            _start_in(b, bufs[b], isems[b])

        def body(p, carry):
            for b in range(NBUF):
                ch = NBUF * p + b
                _wait_in(bufs[b], isems[b])

                @plsc.parallel_loop(0, C, 16, unroll=UNROLL)
                def inner(i, _buf=bufs[b]):
                    process(_buf[pl.ds(i, 16)])
                nxt = ch + NBUF

                @pl.when(nxt < NCH)
                def _():
                    _start_in(nxt, bufs[b], isems[b])
            return carry
        lax.fori_loop(0, NCH // NBUF, body, 0)

    def merge_and_scan(h, nb, kp):
        def lr(g, carry):
            acc = h[0, pl.ds(g * 16, 16)]
            for l in range(1, 16):
                acc = acc + h[l, pl.ds(g * 16, 16)]
            lh[pl.ds(g * 16, 16)] = acc
            return carry
        lax.fori_loop(0, nb // 16, lr, 0)

        pltpu.sync_copy(lh, sp.at[s])
        plsc.subcore_barrier()
        pltpu.sync_copy(sp, h1)
        plsc.subcore_barrier()

        ng = nb // 16

        def scan_body(t, carry):
            cnt, maxfail, above = carry
            g = ng - 1 - t
            m = h1[0, pl.ds(g * 16, 16)]
            for r in range(1, 16):
                m = m + h1[r, pl.ds(g * 16, 16)]
            sfx = jnp.flip(plsc.cumsum(jnp.flip(m))) + above
            ge = sfx >= kp
            cnt = cnt + jnp.sum(jnp.where(ge, 1, 0))
            maxfail = jnp.maximum(maxfail, jnp.max(jnp.where(ge, 0, sfx)))
            above = above + jnp.sum(m)
            return cnt, maxfail, above

        cnt, maxfail, _ = lax.fori_loop(
            0, ng, scan_body, (jnp.int32(0), jnp.int32(0), jnp.int32(0)))
        return cnt - 1, kp - maxfail

    def p1(v):
        bits = _absbits(v)
        plsc.addupdate_scatter(h1, [iota, bits >> SH1], ones16)
    stream_pass(p1)
    b1r, k2 = merge_and_scan(h1, NB1, jnp.int32(KTOP))

    def p2(v):
        bits = _absbits(v)
        pred = (bits >> SH1) == b1r
        binv = jnp.bitwise_and(bits >> SH2, jnp.int32(NB2 - 1))
        plsc.addupdate_scatter(h2, [iota, binv], ones16, mask=pred)
    stream_pass(p2)
    b2r, k3 = merge_and_scan(h2, NB2, k2)
    pfx2 = (b1r << 10) | b2r

    def p3(v):
        bits = _absbits(v)
        pred = (bits >> SH2) == pfx2
        binv = jnp.bitwise_and(bits, jnp.int32(NB3 - 1))
        plsc.addupdate_scatter(h3, [iota, binv], ones16, mask=pred)
    stream_pass(p3)
    b3r, _ = merge_and_scan(h3, NB3, k3)
    thr = (pfx2 << 10) | b3r

    def _start_out(ch, buf, sem):
        pltpu.async_copy(buf, out_hbm.at[pl.ds(base + ch * C, C)], sem)

    def _wait_out(buf, sem):
        pltpu.make_async_copy(buf, out_hbm.at[pl.ds(base, C)], sem).wait()

    for b in range(NBUF):
        _start_in(b, bufs[b], isems[b])

    def mbody(p, carry):
        for b in range(NBUF):
            ch = NBUF * p + b
            _wait_in(bufs[b], isems[b])

            @plsc.parallel_loop(0, C, 16, unroll=UNROLL)
            def inner(i, _buf=bufs[b]):
                bits = _absbits(_buf[pl.ds(i, 16)])
                _buf[pl.ds(i, 16)] = jnp.where(
                    bits >= thr, jnp.int32(_ONE_F32_BITS), jnp.int32(0))
            _start_out(ch, bufs[b], osems[b])
            nxt = ch + NBUF

            @pl.when(nxt < NCH)
            def _():
                _wait_out(bufs[b], osems[b])
                _start_in(nxt, bufs[b], isems[b])
        return carry
    lax.fori_loop(0, NCH // NBUF, mbody, 0)
    for b in range(NBUF):
        _wait_out(bufs[b], osems[b])


def _body(down_hbm, up_hbm, dout_hbm, uout_hbm,
          h1, h2, h3, lh, b0, b1, b2, b3, sp,
          i0, i1, i2, i3, o0, o1, o2, o3):
    c = lax.axis_index("c")
    s = lax.axis_index("s")
    bufs = (b0, b1, b2, b3)
    isems = (i0, i1, i2, i3)
    osems = (o0, o1, o2, o3)

    @pl.when(c == 0)
    def _():
        _run(down_hbm, dout_hbm, s, h1, h2, h3, lh, bufs, sp, isems, osems)

    @pl.when(c == 1)
    def _():
        _run(up_hbm, uout_hbm, s, h1, h2, h3, lh, bufs, sp, isems, osems)


_mesh = plsc.VectorSubcoreMesh(core_axis_name="c", subcore_axis_name="s")

_select = functools.partial(
    pl.kernel,
    out_type=(jax.ShapeDtypeStruct((N,), _i32),
              jax.ShapeDtypeStruct((N,), _i32)),
    mesh=_mesh,
    scratch_types=[
        pltpu.VMEM((16, NB1), _i32),
        pltpu.VMEM((16, NB2), _i32),
        pltpu.VMEM((16, NB3), _i32),
        pltpu.VMEM((NB1,), _i32),
        pltpu.VMEM((C,), _i32),
        pltpu.VMEM((C,), _i32),
        pltpu.VMEM((C,), _i32),
        pltpu.VMEM((C,), _i32),
        pltpu.VMEM_SHARED((16, NB1), _i32),
        pltpu.SemaphoreType.DMA,
        pltpu.SemaphoreType.DMA,
        pltpu.SemaphoreType.DMA,
        pltpu.SemaphoreType.DMA,
        pltpu.SemaphoreType.DMA,
        pltpu.SemaphoreType.DMA,
        pltpu.SemaphoreType.DMA,
        pltpu.SemaphoreType.DMA,
    ],
    compiler_params=pltpu.CompilerParams(
        use_tc_tiling_on_sc=False, needs_layout_passes=False),
)(_body)


def _tile_order_view(x):
    r, c = x.shape
    b = lax.bitcast_convert_type(x, _i32)
    return b.reshape(r // 8, 8, c // 128, 128).transpose(0, 2, 1, 3).reshape(-1)


def _from_tile_order(m, r, c):
    v = m.reshape(r // 8, c // 128, 8, 128).transpose(0, 2, 1, 3).reshape(r, c)
    return lax.bitcast_convert_type(v, _f32)


@jax.jit
def kernel(down_mask, up_mask):
    d, u = _select(_tile_order_view(down_mask), _tile_order_view(up_mask))
    return (_from_tile_order(d, *down_mask.shape),
            _from_tile_order(u, *up_mask.shape))
